# Initial kernel scaffold; baseline (speedup 1.0000x reference)
import jax
import jax.numpy as jnp
from jax.experimental import pallas as pl


def kernel(x, edge_logits):
    raise NotImplementedError("write your pallas kernel here")


# speedup vs baseline: 2.1574x; 2.1574x over previous
import jax
import jax.numpy as jnp
from jax.experimental import pallas as pl
from jax.experimental.pallas import tpu as pltpu

_IN = 1024
_OUT = 1023
_T = 3.0


def _body(x_ref, el_ref, o_ref):
    el = el_ref[...]
    soft = jax.nn.softmax(el / _T, axis=1)
    m = jnp.max(soft, axis=1, keepdims=True)
    iota = jax.lax.broadcasted_iota(jnp.int32, (_IN, _OUT), 1)
    route = jnp.min(jnp.where(soft == m, iota, _OUT), axis=1, keepdims=True)
    w = (route == iota).astype(jnp.float32)
    num = jnp.dot(x_ref[...], w, preferred_element_type=jnp.float32)
    den = jnp.sum(w, axis=0)
    o_ref[...] = jnp.clip(num / (den + 1e-12), -10000.0, 10000.0)


def kernel(x, edge_logits):
    B = x.shape[0]
    blk = 1024
    return pl.pallas_call(
        _body,
        grid=(B // blk,),
        in_specs=[
            pl.BlockSpec((blk, _IN), lambda i: (i, 0)),
            pl.BlockSpec((_IN, _OUT), lambda i: (0, 0)),
        ],
        out_specs=pl.BlockSpec((blk, _OUT), lambda i: (i, 0)),
        out_shape=jax.ShapeDtypeStruct((B, _OUT), jnp.float32),
    )(x, edge_logits)
